# Initial kernel scaffold; baseline (speedup 1.0000x reference)
import jax
import jax.numpy as jnp
from jax.experimental import pallas as pl


def kernel(x):
    raise NotImplementedError("write your pallas kernel here")


# speedup vs baseline: 2.0951x; 2.0951x over previous
import functools

import jax
import jax.numpy as jnp
from jax.experimental import pallas as pl
from jax.experimental.pallas import tpu as pltpu


def _cumsum_kernel(x_ref, o_ref, carry_ref, *, bc):
    j = pl.program_id(1)

    @pl.when(j == 0)
    def _():
        carry_ref[...] = jnp.zeros_like(carry_ref)

    x = x_ref[...]
    row = jax.lax.broadcasted_iota(jnp.int32, (bc, bc), 0)
    col = jax.lax.broadcasted_iota(jnp.int32, (bc, bc), 1)
    tri = (row <= col).astype(jnp.float32)
    ps = jax.lax.dot(x, tri, precision=jax.lax.Precision.HIGHEST)
    carry = carry_ref[:, :1]
    o_ref[...] = ps + carry
    carry_ref[...] = jnp.broadcast_to(carry + ps[:, -1:], carry_ref.shape)


def kernel(x):
    m, n = x.shape
    br, bc = 512, 256
    grid = (m // br, n // bc)
    return pl.pallas_call(
        functools.partial(_cumsum_kernel, bc=bc),
        grid=grid,
        in_specs=[pl.BlockSpec((br, bc), lambda i, j: (i, j))],
        out_specs=pl.BlockSpec((br, bc), lambda i, j: (i, j)),
        out_shape=jax.ShapeDtypeStruct((m, n), x.dtype),
        scratch_shapes=[pltpu.VMEM((br, 128), jnp.float32)],
    )(x)


# speedup vs baseline: 4.4207x; 2.1099x over previous
import functools

import jax
import jax.numpy as jnp
from jax.experimental import pallas as pl
from jax.experimental.pallas import tpu as pltpu


def _cumsum_kernel(x_ref, o_ref, carry_ref, *, nchunks):
    j = pl.program_id(1)

    @pl.when(j == 0)
    def _():
        carry_ref[...] = jnp.zeros_like(carry_ref)

    row = jax.lax.broadcasted_iota(jnp.int32, (128, 128), 0)
    col = jax.lax.broadcasted_iota(jnp.int32, (128, 128), 1)
    tri = (row <= col).astype(jnp.bfloat16)

    off = carry_ref[:, :1]
    for c in range(nchunks):
        xc = x_ref[:, c * 128:(c + 1) * 128]
        ps = jax.lax.dot(xc.astype(jnp.bfloat16), tri,
                         preferred_element_type=jnp.float32)
        o_ref[:, c * 128:(c + 1) * 128] = ps + off
        off = off + ps[:, -1:]
    carry_ref[...] = jnp.broadcast_to(off, carry_ref.shape)


def kernel(x):
    m, n = x.shape
    br, bc = 512, 1024
    grid = (m // br, n // bc)
    return pl.pallas_call(
        functools.partial(_cumsum_kernel, nchunks=bc // 128),
        grid=grid,
        in_specs=[pl.BlockSpec((br, bc), lambda i, j: (i, j))],
        out_specs=pl.BlockSpec((br, bc), lambda i, j: (i, j)),
        out_shape=jax.ShapeDtypeStruct((m, n), x.dtype),
        scratch_shapes=[pltpu.VMEM((br, 128), jnp.float32)],
        compiler_params=pltpu.CompilerParams(
            dimension_semantics=("parallel", "arbitrary")),
    )(x)


# speedup vs baseline: 5.2491x; 1.1874x over previous
import functools

import jax
import jax.numpy as jnp
from jax.experimental import pallas as pl
from jax.experimental.pallas import tpu as pltpu


def _cumsum_kernel(x_ref, o_ref, carry_ref, *, nchunks):
    j = pl.program_id(1)

    @pl.when(j == 0)
    def _():
        carry_ref[...] = jnp.zeros_like(carry_ref)

    row = jax.lax.broadcasted_iota(jnp.int32, (128, 128), 0)
    col = jax.lax.broadcasted_iota(jnp.int32, (128, 128), 1)
    tri = (row <= col).astype(jnp.bfloat16)
    ones = jnp.ones((128, 128), jnp.bfloat16)

    off = carry_ref[...]
    for c in range(nchunks):
        xc = x_ref[:, c * 128:(c + 1) * 128].astype(jnp.bfloat16)
        ps = jax.lax.dot(xc, tri, preferred_element_type=jnp.float32)
        tot = jax.lax.dot(xc, ones, preferred_element_type=jnp.float32)
        o_ref[:, c * 128:(c + 1) * 128] = ps + off
        off = off + tot
    carry_ref[...] = off


def kernel(x):
    m, n = x.shape
    br, bc = 512, 1024
    grid = (m // br, n // bc)
    return pl.pallas_call(
        functools.partial(_cumsum_kernel, nchunks=bc // 128),
        grid=grid,
        in_specs=[pl.BlockSpec((br, bc), lambda i, j: (i, j))],
        out_specs=pl.BlockSpec((br, bc), lambda i, j: (i, j)),
        out_shape=jax.ShapeDtypeStruct((m, n), x.dtype),
        scratch_shapes=[pltpu.VMEM((br, 128), jnp.float32)],
        compiler_params=pltpu.CompilerParams(
            dimension_semantics=("parallel", "arbitrary")),
    )(x)


# speedup vs baseline: 6.3384x; 1.2075x over previous
import functools

import jax
import jax.numpy as jnp
from jax.experimental import pallas as pl
from jax.experimental.pallas import tpu as pltpu


def _cumsum_kernel(x_ref, o_ref, carry_ref, *, nchunks):
    j = pl.program_id(1)

    @pl.when(j == 0)
    def _():
        carry_ref[...] = jnp.zeros_like(carry_ref)

    row = jax.lax.broadcasted_iota(jnp.int32, (128, 128), 0)
    col = jax.lax.broadcasted_iota(jnp.int32, (128, 128), 1)
    tri = (row <= col).astype(jnp.bfloat16)
    ones = jnp.ones((128, 128), jnp.bfloat16)

    off = carry_ref[...]
    for c in range(nchunks):
        xc = x_ref[:, c * 128:(c + 1) * 128].astype(jnp.bfloat16)
        ps = jax.lax.dot(xc, tri, preferred_element_type=jnp.float32)
        tot = jax.lax.dot(xc, ones, preferred_element_type=jnp.float32)
        o_ref[:, c * 128:(c + 1) * 128] = ps + off
        off = off + tot
    carry_ref[...] = off


def kernel(x):
    m, n = x.shape
    br, bc = 512, 2048
    grid = (m // br, n // bc)
    return pl.pallas_call(
        functools.partial(_cumsum_kernel, nchunks=bc // 128),
        grid=grid,
        in_specs=[pl.BlockSpec((br, bc), lambda i, j: (i, j))],
        out_specs=pl.BlockSpec((br, bc), lambda i, j: (i, j)),
        out_shape=jax.ShapeDtypeStruct((m, n), x.dtype),
        scratch_shapes=[pltpu.VMEM((br, 128), jnp.float32)],
        compiler_params=pltpu.CompilerParams(
            dimension_semantics=("parallel", "arbitrary")),
    )(x)


# speedup vs baseline: 6.5414x; 1.0320x over previous
import functools

import jax
import jax.numpy as jnp
from jax.experimental import pallas as pl
from jax.experimental.pallas import tpu as pltpu


def _cumsum_kernel(x_ref, o_ref, carry_ref, *, nchunks):
    j = pl.program_id(1)

    @pl.when(j == 0)
    def _():
        carry_ref[...] = jnp.zeros_like(carry_ref)

    row = jax.lax.broadcasted_iota(jnp.int32, (128, 128), 0)
    col = jax.lax.broadcasted_iota(jnp.int32, (128, 128), 1)
    tri = (row <= col).astype(jnp.bfloat16)
    ones = jnp.ones((128, 128), jnp.bfloat16)

    off = carry_ref[...]
    for c in range(nchunks):
        xc = x_ref[:, c * 128:(c + 1) * 128].astype(jnp.bfloat16)
        ps = jax.lax.dot(xc, tri, preferred_element_type=jnp.float32)
        tot = jax.lax.dot(xc, ones, preferred_element_type=jnp.float32)
        o_ref[:, c * 128:(c + 1) * 128] = ps + off
        off = off + tot
    carry_ref[...] = off


def kernel(x):
    m, n = x.shape
    br, bc = 512, 4096
    grid = (m // br, n // bc)
    return pl.pallas_call(
        functools.partial(_cumsum_kernel, nchunks=bc // 128),
        grid=grid,
        in_specs=[pl.BlockSpec((br, bc), lambda i, j: (i, j))],
        out_specs=pl.BlockSpec((br, bc), lambda i, j: (i, j)),
        out_shape=jax.ShapeDtypeStruct((m, n), x.dtype),
        scratch_shapes=[pltpu.VMEM((br, 128), jnp.float32)],
        compiler_params=pltpu.CompilerParams(
            dimension_semantics=("parallel", "arbitrary")),
    )(x)


# speedup vs baseline: 6.5557x; 1.0022x over previous
import functools

import jax
import jax.numpy as jnp
from jax.experimental import pallas as pl
from jax.experimental.pallas import tpu as pltpu


def _cumsum_kernel(x_ref, o_ref, carry_ref, *, nchunks):
    j = pl.program_id(1)

    @pl.when(j == 0)
    def _():
        carry_ref[...] = jnp.zeros_like(carry_ref)

    row = jax.lax.broadcasted_iota(jnp.int32, (128, 128), 0)
    col = jax.lax.broadcasted_iota(jnp.int32, (128, 128), 1)
    tri = (row <= col).astype(jnp.bfloat16)
    ones = jnp.ones((128, 128), jnp.bfloat16)

    off = carry_ref[...]
    for c in range(nchunks):
        xc = x_ref[:, c * 128:(c + 1) * 128].astype(jnp.bfloat16)
        ps = jax.lax.dot(xc, tri, preferred_element_type=jnp.float32)
        tot = jax.lax.dot(xc, ones, preferred_element_type=jnp.float32)
        o_ref[:, c * 128:(c + 1) * 128] = ps + off
        off = off + tot
    carry_ref[...] = off


def kernel(x):
    m, n = x.shape
    br, bc = 256, 8192
    grid = (m // br, n // bc)
    return pl.pallas_call(
        functools.partial(_cumsum_kernel, nchunks=bc // 128),
        grid=grid,
        in_specs=[pl.BlockSpec((br, bc), lambda i, j: (i, j))],
        out_specs=pl.BlockSpec((br, bc), lambda i, j: (i, j)),
        out_shape=jax.ShapeDtypeStruct((m, n), x.dtype),
        scratch_shapes=[pltpu.VMEM((br, 128), jnp.float32)],
        compiler_params=pltpu.CompilerParams(
            dimension_semantics=("parallel", "arbitrary")),
    )(x)
